# baseline (speedup 1.0000x reference)
import functools

import jax
import jax.numpy as jnp
from jax import lax
from jax.experimental import pallas as pl
from jax.experimental.pallas import tpu as pltpu
from jax.experimental.pallas import tpu_sc as plsc

_DIM = 32
_NE = 8192
_TBLK = 256
_PAD = 128


def _dist_body_dual(x_ref, e_ref, da_ref, db_ref, ind_ref, diff_ref, acc_ref):
    _dist_common(x_ref, e_ref, (da_ref, db_ref), ind_ref, diff_ref, acc_ref)


def _dist_body_single(x_ref, e_ref, da_ref, ind_ref, diff_ref, acc_ref):
    _dist_common(x_ref, e_ref, (da_ref,), ind_ref, diff_ref, acc_ref)


def _dist_common(x_ref, e_ref, dist_refs, ind_ref, diff_ref, acc_ref):
    t = pl.program_id(0)
    x = x_ref[...]
    e = e_ref[...]
    xsq = jnp.sum(x * x, axis=1, keepdims=True)
    esq = jnp.sum(e * e, axis=0, keepdims=True)
    m = jnp.dot(x, e, preferred_element_type=jnp.float32)
    dist = xsq - 2.0 * m + esq
    for r in dist_refs:
        r[...] = dist
    minval = jnp.min(dist, axis=1, keepdims=True)
    idx = lax.broadcasted_iota(jnp.int32, dist.shape, 1)
    cand = jnp.where(dist == minval, idx, jnp.int32(_NE))
    ind_ref[...] = jnp.min(cand, axis=1, keepdims=True)

    @pl.when(t == 0)
    def _():
        acc_ref[0, 0] = 0.0

    acc_ref[0, 0] += jnp.sum(minval)

    @pl.when(t == pl.num_programs(0) - 1)
    def _():
        denom = _TBLK * pl.num_programs(0) * _DIM
        diff_ref[...] = jnp.full((1, 1), acc_ref[0, 0] / denom, jnp.float32)


def _dist_argmin(flat, embed, ndist):
    n = flat.shape[0]
    dist_sd = jax.ShapeDtypeStruct((n, _NE), jnp.float32)
    dist_spec = pl.BlockSpec((_TBLK, _NE), lambda t: (t, 0))
    body = _dist_body_dual if ndist == 2 else _dist_body_single
    return pl.pallas_call(
        body,
        grid=(n // _TBLK,),
        in_specs=[
            pl.BlockSpec((_TBLK, _DIM), lambda t: (t, 0)),
            pl.BlockSpec((_DIM, _NE), lambda t: (0, 0)),
        ],
        out_specs=[dist_spec] * ndist + [
            pl.BlockSpec((_TBLK, 1), lambda t: (t, 0)),
            pl.BlockSpec((1, 1), lambda t: (0, 0)),
        ],
        out_shape=[dist_sd] * ndist + [
            jax.ShapeDtypeStruct((n, 1), jnp.int32),
            jax.ShapeDtypeStruct((1, 1), jnp.float32),
        ],
        scratch_shapes=[pltpu.SMEM((1, 1), jnp.float32)],
    )(flat, embed)


_CROWS = 8
_CCOLS = 4096


def _sc_copy(src):
    n = src.shape[0]
    info = plsc.get_sparse_core_info()
    nw = info.num_cores * info.num_subcores
    rpw = n // nw
    nchunks = (rpw // _CROWS) * (_NE // _CCOLS)
    ncols = _NE // _CCOLS
    mesh = plsc.VectorSubcoreMesh(core_axis_name="c", subcore_axis_name="s")

    @functools.partial(
        pl.kernel,
        mesh=mesh,
        out_type=jax.ShapeDtypeStruct((n, _NE), jnp.float32),
        scratch_types=[
            pltpu.VMEM((_CROWS, _CCOLS), jnp.float32),
            pltpu.VMEM((_CROWS, _CCOLS), jnp.float32),
            pltpu.SemaphoreType.DMA,
            pltpu.SemaphoreType.DMA,
            pltpu.SemaphoreType.DMA,
            pltpu.SemaphoreType.DMA,
        ],
    )
    def copy_k(s_ref, d_ref, b0, b1, si0, si1, so0, so1):
        wid = lax.axis_index("s") * info.num_cores + lax.axis_index("c")
        base = wid * rpw

        def chunk(c):
            r = base + (c // ncols) * _CROWS
            col = (c % ncols) * _CCOLS
            return (pl.ds(r, _CROWS), pl.ds(col, _CCOLS))

        bufs = (b0, b1)
        sin = (si0, si1)
        sout = (so0, so1)
        h_in = {}
        h_out = {}
        h_in[0] = pltpu.async_copy(s_ref.at[chunk(0)], bufs[0], sin[0])
        for c in range(nchunks):
            p = c % 2
            h_in[c].wait()
            if c + 1 < nchunks:
                if c - 1 >= 0:
                    h_out[c - 1].wait()
                h_in[c + 1] = pltpu.async_copy(
                    s_ref.at[chunk(c + 1)], bufs[1 - p], sin[1 - p])
            h_out[c] = pltpu.async_copy(bufs[p], d_ref.at[chunk(c)], sout[p])
        h_out[nchunks - 2].wait()
        h_out[nchunks - 1].wait()

    return copy_k(src)


def _sc_gather(codebook_padded, idx_hr, idx_lr):
    n = idx_hr.shape[0]
    info = plsc.get_sparse_core_info()
    nw = info.num_cores * info.num_subcores
    bpw = n // nw
    mesh = plsc.VectorSubcoreMesh(core_axis_name="c", subcore_axis_name="s")
    out = jax.ShapeDtypeStruct((n, _PAD), jnp.float32)

    @functools.partial(
        pl.kernel,
        mesh=mesh,
        out_type=[out, out, out, out],
        scratch_types=[
            pltpu.VMEM((bpw,), jnp.int32),
            pltpu.VMEM((bpw,), jnp.int32),
            pltpu.VMEM((bpw, _PAD), jnp.float32),
            pltpu.VMEM((bpw, _PAD), jnp.float32),
            pltpu.SemaphoreType.DMA,
            pltpu.SemaphoreType.DMA,
        ],
    )
    def gather(table, ihr, ilr, qhr_a, qhr_b, qlr_a, qlr_b,
               iv1, iv2, rv1, rv2, sem1, sem2):
        wid = lax.axis_index("s") * info.num_cores + lax.axis_index("c")
        base = wid * bpw
        pltpu.sync_copy(ihr.at[pl.ds(base, bpw)], iv1)
        pltpu.sync_copy(ilr.at[pl.ds(base, bpw)], iv2)
        cp1 = pltpu.async_copy(table.at[iv1], rv1, sem1)
        cp2 = pltpu.async_copy(table.at[iv2], rv2, sem2)
        cp1.wait()
        cp2.wait()
        pltpu.sync_copy(rv1, qhr_a.at[pl.ds(base, bpw)])
        pltpu.sync_copy(rv1, qhr_b.at[pl.ds(base, bpw)])
        pltpu.sync_copy(rv2, qlr_a.at[pl.ds(base, bpw)])
        pltpu.sync_copy(rv2, qlr_b.at[pl.ds(base, bpw)])

    return gather(codebook_padded, idx_hr, idx_lr)


def kernel(input_hr, input_lr, embed_lr, embed_hr):
    shape3 = input_hr.shape
    shape2 = shape3[:-1]
    flat_hr = input_hr.reshape(-1, _DIM)
    flat_lr = input_lr.reshape(-1, _DIM)

    da_hr, ind_hr, diff_hr = _dist_argmin(flat_hr, embed_lr, ndist=1)
    da_lr, db_lr, ind_lr, diff_lr = _dist_argmin(flat_lr, embed_lr, ndist=2)
    db_hr = _sc_copy(da_hr)

    codebook = jnp.pad(embed_lr.T, ((0, 0), (0, _PAD - _DIM)))
    qhr_a, qhr_b, qlr_a, qlr_b = _sc_gather(
        codebook, ind_hr.reshape(-1), ind_lr.reshape(-1))

    d_hr = diff_hr.reshape(())
    d_lr = diff_lr.reshape(())
    i_hr = ind_hr.reshape(shape2)
    i_lr = ind_lr.reshape(shape2)

    return (qhr_a[:, :_DIM].reshape(shape3), qlr_a[:, :_DIM].reshape(shape3),
            qhr_b[:, :_DIM].reshape(shape3), qlr_b[:, :_DIM].reshape(shape3),
            d_hr, d_lr, d_hr, d_lr,
            i_hr, i_lr, i_hr, i_lr,
            da_hr, da_lr, db_hr, db_lr)


# speedup vs baseline: 1.2615x; 1.2615x over previous
import functools

import jax
import jax.numpy as jnp
from jax import lax
from jax.experimental import pallas as pl
from jax.experimental.pallas import tpu as pltpu
from jax.experimental.pallas import tpu_sc as plsc

_DIM = 32
_NE = 8192
_TBLK = 128
_PAD = 128


def _dist_body(x_ref, e_ref, da_hr_ref, db_hr_ref, da_lr_ref, db_lr_ref,
               ind_hr_ref, ind_lr_ref, qa_hr_ref, qb_hr_ref, qa_lr_ref,
               qb_lr_ref, diff_ref, acc_ref):
    t = pl.program_id(0)
    x = x_ref[...]
    e = e_ref[...]
    xsq = jnp.sum(x * x, axis=1, keepdims=True)
    esq = jnp.sum(e * e, axis=0, keepdims=True)
    m = jnp.dot(x, e, preferred_element_type=jnp.float32)
    dist = xsq - 2.0 * m + esq
    da_hr_ref[...] = dist[:_TBLK]
    db_hr_ref[...] = dist[:_TBLK]
    da_lr_ref[...] = dist[_TBLK:]
    db_lr_ref[...] = dist[_TBLK:]
    minval = jnp.min(dist, axis=1, keepdims=True)
    idx = lax.broadcasted_iota(jnp.int32, dist.shape, 1)
    cand = jnp.where(dist == minval, idx, jnp.int32(_NE))
    ind = jnp.min(cand, axis=1, keepdims=True)
    ind_hr_ref[...] = ind[:_TBLK]
    ind_lr_ref[...] = ind[_TBLK:]
    onehot = jnp.where(idx == ind, 1.0, 0.0).astype(jnp.float32)
    q = lax.dot_general(onehot, e, (((1,), (1,)), ((), ())),
                        preferred_element_type=jnp.float32)
    qa_hr_ref[...] = q[:_TBLK]
    qb_hr_ref[...] = q[:_TBLK]
    qa_lr_ref[...] = q[_TBLK:]
    qb_lr_ref[...] = q[_TBLK:]

    @pl.when(t == 0)
    def _():
        acc_ref[0, 0] = 0.0
        acc_ref[0, 1] = 0.0

    acc_ref[0, 0] += jnp.sum(minval[:_TBLK])
    acc_ref[0, 1] += jnp.sum(minval[_TBLK:])

    @pl.when(t == pl.num_programs(0) - 1)
    def _():
        denom = _TBLK * pl.num_programs(0) * _DIM
        diff_ref[...] = jnp.concatenate(
            [jnp.full((1, 1), acc_ref[0, 0] / denom, jnp.float32),
             jnp.full((1, 1), acc_ref[0, 1] / denom, jnp.float32)], axis=1)


def _dist_argmin(flat_hr, flat_lr, embed):
    n = flat_hr.shape[0]
    nblk = n // _TBLK
    x_il = jnp.concatenate(
        [flat_hr.reshape(nblk, _TBLK, _DIM), flat_lr.reshape(nblk, _TBLK, _DIM)],
        axis=1).reshape(2 * n, _DIM)
    dist_sd = jax.ShapeDtypeStruct((n, _NE), jnp.float32)
    ind_sd = jax.ShapeDtypeStruct((n, 1), jnp.int32)
    return pl.pallas_call(
        _dist_body,
        grid=(nblk,),
        in_specs=[
            pl.BlockSpec((2 * _TBLK, _DIM), lambda t: (t, 0)),
            pl.BlockSpec((_DIM, _NE), lambda t: (0, 0)),
        ],
        out_specs=[
            pl.BlockSpec((_TBLK, _NE), lambda t: (t, 0)),
            pl.BlockSpec((_TBLK, _NE), lambda t: (t, 0)),
            pl.BlockSpec((_TBLK, _NE), lambda t: (t, 0)),
            pl.BlockSpec((_TBLK, _NE), lambda t: (t, 0)),
            pl.BlockSpec((_TBLK, 1), lambda t: (t, 0)),
            pl.BlockSpec((_TBLK, 1), lambda t: (t, 0)),
            pl.BlockSpec((_TBLK, _DIM), lambda t: (t, 0)),
            pl.BlockSpec((_TBLK, _DIM), lambda t: (t, 0)),
            pl.BlockSpec((_TBLK, _DIM), lambda t: (t, 0)),
            pl.BlockSpec((_TBLK, _DIM), lambda t: (t, 0)),
            pl.BlockSpec((1, 2), lambda t: (0, 0)),
        ],
        out_shape=[dist_sd, dist_sd, dist_sd, dist_sd, ind_sd, ind_sd,
                   jax.ShapeDtypeStruct((n, _DIM), jnp.float32),
                   jax.ShapeDtypeStruct((n, _DIM), jnp.float32),
                   jax.ShapeDtypeStruct((n, _DIM), jnp.float32),
                   jax.ShapeDtypeStruct((n, _DIM), jnp.float32),
                   jax.ShapeDtypeStruct((1, 2), jnp.float32)],
        scratch_shapes=[pltpu.SMEM((1, 2), jnp.float32)],
    )(x_il, embed)


def _sc_gather(codebook_padded, idx_hr, idx_lr):
    n = idx_hr.shape[0]
    info = plsc.get_sparse_core_info()
    nw = info.num_cores * info.num_subcores
    bpw = n // nw
    mesh = plsc.VectorSubcoreMesh(core_axis_name="c", subcore_axis_name="s")
    out = jax.ShapeDtypeStruct((n, _PAD), jnp.float32)

    @functools.partial(
        pl.kernel,
        mesh=mesh,
        out_type=[out, out, out, out],
        scratch_types=[
            pltpu.VMEM((bpw,), jnp.int32),
            pltpu.VMEM((bpw,), jnp.int32),
            pltpu.VMEM((bpw, _PAD), jnp.float32),
            pltpu.VMEM((bpw, _PAD), jnp.float32),
            pltpu.SemaphoreType.DMA,
            pltpu.SemaphoreType.DMA,
        ],
    )
    def gather(table, ihr, ilr, qhr_a, qhr_b, qlr_a, qlr_b,
               iv1, iv2, rv1, rv2, sem1, sem2):
        wid = lax.axis_index("s") * info.num_cores + lax.axis_index("c")
        base = wid * bpw
        pltpu.sync_copy(ihr.at[pl.ds(base, bpw)], iv1)
        pltpu.sync_copy(ilr.at[pl.ds(base, bpw)], iv2)
        cp1 = pltpu.async_copy(table.at[iv1], rv1, sem1)
        cp2 = pltpu.async_copy(table.at[iv2], rv2, sem2)
        cp1.wait()
        cp2.wait()
        pltpu.sync_copy(rv1, qhr_a.at[pl.ds(base, bpw)])
        pltpu.sync_copy(rv1, qhr_b.at[pl.ds(base, bpw)])
        pltpu.sync_copy(rv2, qlr_a.at[pl.ds(base, bpw)])
        pltpu.sync_copy(rv2, qlr_b.at[pl.ds(base, bpw)])

    return gather(codebook_padded, idx_hr, idx_lr)


def kernel(input_hr, input_lr, embed_lr, embed_hr):
    shape3 = input_hr.shape
    shape2 = shape3[:-1]
    flat_hr = input_hr.reshape(-1, _DIM)
    flat_lr = input_lr.reshape(-1, _DIM)

    (da_hr, db_hr, da_lr, db_lr, ind_hr, ind_lr,
     qhr_a, qhr_b, qlr_a, qlr_b, diff) = _dist_argmin(flat_hr, flat_lr, embed_lr)

    d_hr = diff[0, 0]
    d_lr = diff[0, 1]
    i_hr = ind_hr.reshape(shape2)
    i_lr = ind_lr.reshape(shape2)

    return (qhr_a.reshape(shape3), qlr_a.reshape(shape3),
            qhr_b.reshape(shape3), qlr_b.reshape(shape3),
            d_hr, d_lr, d_hr, d_lr,
            i_hr, i_lr, i_hr, i_lr,
            da_hr, da_lr, db_hr, db_lr)


# speedup vs baseline: 1.3095x; 1.0380x over previous
import functools

import jax
import jax.numpy as jnp
from jax import lax
from jax.experimental import pallas as pl
from jax.experimental.pallas import tpu as pltpu
from jax.experimental.pallas import tpu_sc as plsc

_DIM = 32
_NE = 8192
_TBLK = 128
_PAD = 128


def _dist_body(xh_ref, xl_ref, e_ref, da_hr_ref, db_hr_ref, da_lr_ref,
               db_lr_ref, ind_hr_ref, ind_lr_ref, diff_ref, acc_ref):
    t = pl.program_id(0)
    x = jnp.concatenate([xh_ref[...], xl_ref[...]], axis=0)
    e = e_ref[...]
    xsq = jnp.sum(x * x, axis=1, keepdims=True)
    esq = jnp.sum(e * e, axis=0, keepdims=True)
    m = jnp.dot(x, e, preferred_element_type=jnp.float32)
    dist = xsq - 2.0 * m + esq
    da_hr_ref[...] = dist[:_TBLK]
    db_hr_ref[...] = dist[:_TBLK]
    da_lr_ref[...] = dist[_TBLK:]
    db_lr_ref[...] = dist[_TBLK:]
    minval = jnp.min(dist, axis=1, keepdims=True)
    idx = lax.broadcasted_iota(jnp.int32, dist.shape, 1)
    cand = jnp.where(dist == minval, idx, jnp.int32(_NE))
    ind = jnp.min(cand, axis=1, keepdims=True)
    ind_hr_ref[...] = ind[:_TBLK]
    ind_lr_ref[...] = ind[_TBLK:]

    @pl.when(t == 0)
    def _():
        acc_ref[0, 0] = 0.0
        acc_ref[0, 1] = 0.0

    acc_ref[0, 0] += jnp.sum(minval[:_TBLK])
    acc_ref[0, 1] += jnp.sum(minval[_TBLK:])

    @pl.when(t == pl.num_programs(0) - 1)
    def _():
        denom = _TBLK * pl.num_programs(0) * _DIM
        diff_ref[...] = jnp.concatenate(
            [jnp.full((1, 1), acc_ref[0, 0] / denom, jnp.float32),
             jnp.full((1, 1), acc_ref[0, 1] / denom, jnp.float32)], axis=1)


def _dist_argmin(flat_hr, flat_lr, embed):
    n = flat_hr.shape[0]
    nblk = n // _TBLK
    dist_sd = jax.ShapeDtypeStruct((n, _NE), jnp.float32)
    ind_sd = jax.ShapeDtypeStruct((n, 1), jnp.int32)
    return pl.pallas_call(
        _dist_body,
        grid=(nblk,),
        in_specs=[
            pl.BlockSpec((_TBLK, _DIM), lambda t: (t, 0)),
            pl.BlockSpec((_TBLK, _DIM), lambda t: (t, 0)),
            pl.BlockSpec((_DIM, _NE), lambda t: (0, 0)),
        ],
        out_specs=[
            pl.BlockSpec((_TBLK, _NE), lambda t: (t, 0)),
            pl.BlockSpec((_TBLK, _NE), lambda t: (t, 0)),
            pl.BlockSpec((_TBLK, _NE), lambda t: (t, 0)),
            pl.BlockSpec((_TBLK, _NE), lambda t: (t, 0)),
            pl.BlockSpec((_TBLK, 1), lambda t: (t, 0)),
            pl.BlockSpec((_TBLK, 1), lambda t: (t, 0)),
            pl.BlockSpec((1, 2), lambda t: (0, 0)),
        ],
        out_shape=[dist_sd, dist_sd, dist_sd, dist_sd, ind_sd, ind_sd,
                   jax.ShapeDtypeStruct((1, 2), jnp.float32)],
        scratch_shapes=[pltpu.SMEM((1, 2), jnp.float32)],
    )(flat_hr, flat_lr, embed)


def _sc_gather(codebook_padded, idx_hr, idx_lr):
    n = idx_hr.shape[0]
    info = plsc.get_sparse_core_info()
    nw = info.num_cores * info.num_subcores
    bpw = n // nw
    mesh = plsc.VectorSubcoreMesh(core_axis_name="c", subcore_axis_name="s")
    out = jax.ShapeDtypeStruct((n, _PAD), jnp.float32)

    @functools.partial(
        pl.kernel,
        mesh=mesh,
        out_type=[out, out, out, out],
        scratch_types=[
            pltpu.VMEM((bpw,), jnp.int32),
            pltpu.VMEM((bpw,), jnp.int32),
            pltpu.VMEM((bpw, _PAD), jnp.float32),
            pltpu.VMEM((bpw, _PAD), jnp.float32),
            pltpu.SemaphoreType.DMA,
            pltpu.SemaphoreType.DMA,
        ],
    )
    def gather(table, ihr, ilr, qhr_a, qhr_b, qlr_a, qlr_b,
               iv1, iv2, rv1, rv2, sem1, sem2):
        wid = lax.axis_index("s") * info.num_cores + lax.axis_index("c")
        base = wid * bpw
        pltpu.sync_copy(ihr.at[pl.ds(base, bpw)], iv1)
        pltpu.sync_copy(ilr.at[pl.ds(base, bpw)], iv2)
        cp1 = pltpu.async_copy(table.at[iv1], rv1, sem1)
        cp2 = pltpu.async_copy(table.at[iv2], rv2, sem2)
        cp1.wait()
        cp2.wait()
        pltpu.sync_copy(rv1, qhr_a.at[pl.ds(base, bpw)])
        pltpu.sync_copy(rv1, qhr_b.at[pl.ds(base, bpw)])
        pltpu.sync_copy(rv2, qlr_a.at[pl.ds(base, bpw)])
        pltpu.sync_copy(rv2, qlr_b.at[pl.ds(base, bpw)])

    return gather(codebook_padded, idx_hr, idx_lr)


def kernel(input_hr, input_lr, embed_lr, embed_hr):
    shape3 = input_hr.shape
    shape2 = shape3[:-1]
    flat_hr = input_hr.reshape(-1, _DIM)
    flat_lr = input_lr.reshape(-1, _DIM)

    (da_hr, db_hr, da_lr, db_lr,
     ind_hr, ind_lr, diff) = _dist_argmin(flat_hr, flat_lr, embed_lr)

    codebook = jnp.pad(embed_lr.T, ((0, 0), (0, _PAD - _DIM)))
    qhr_a, qhr_b, qlr_a, qlr_b = _sc_gather(
        codebook, ind_hr.reshape(-1), ind_lr.reshape(-1))

    d_hr = diff[0, 0]
    d_lr = diff[0, 1]
    i_hr = ind_hr.reshape(shape2)
    i_lr = ind_lr.reshape(shape2)

    return (qhr_a[:, :_DIM].reshape(shape3), qlr_a[:, :_DIM].reshape(shape3),
            qhr_b[:, :_DIM].reshape(shape3), qlr_b[:, :_DIM].reshape(shape3),
            d_hr, d_lr, d_hr, d_lr,
            i_hr, i_lr, i_hr, i_lr,
            da_hr, da_lr, db_hr, db_lr)
